# Initial kernel scaffold; baseline (speedup 1.0000x reference)
import jax
import jax.numpy as jnp
from jax.experimental import pallas as pl


def kernel(x, edge_index, W1f, b1f, W2f, b2f, W1p, b1p, W2p, b2p):
    raise NotImplementedError("write your pallas kernel here")


# speedup vs baseline: 4.0776x; 4.0776x over previous
import functools

import jax
import jax.numpy as jnp
from jax import lax
from jax.experimental import pallas as pl
from jax.experimental.pallas import tpu as pltpu
from jax.experimental.pallas import tpu_sc as plsc

N = 10000
E = 320000
D_IN = 128
D_OUT = 128
A_DIM = 256

NC = 2
NS = 16
NW = NC * NS
NP = 10240
ZR = NP // NS
CH = 80

_SC_MESH = plsc.VectorSubcoreMesh(core_axis_name="c", subcore_axis_name="s")


@functools.partial(
    pl.kernel,
    out_type=[
        jax.ShapeDtypeStruct((NC, NP, D_IN), jnp.float32),
        jax.ShapeDtypeStruct((NC, NP), jnp.float32),
    ],
    mesh=_SC_MESH,
    scratch_types=[
        pltpu.VMEM((CH,), jnp.int32),
        pltpu.VMEM((CH,), jnp.int32),
        pltpu.VMEM((CH, D_IN), jnp.float32),
        pltpu.VMEM((CH,), jnp.float32),
        pltpu.VMEM_SHARED((NP, D_IN), jnp.float32),
        pltpu.VMEM_SHARED((NP,), jnp.float32),
        pltpu.SemaphoreType.DMA,
    ],
)
def _sc_gin_agg(x_hbm, src_hbm, dst_hbm, z_hbm, z1_hbm, ones_hbm,
                agg_hbm, deg_hbm,
                idx_s, idx_d, rows, ones_v, acc, dacc, sem):
    c = lax.axis_index("c")
    s = lax.axis_index("s")
    w = c * NS + s
    epw = E // NW
    nchunks = epw // CH

    pltpu.sync_copy(z_hbm, acc.at[pl.ds(s * ZR, ZR)])
    pltpu.sync_copy(z1_hbm, dacc.at[pl.ds(s * ZR, ZR)])
    pltpu.sync_copy(ones_hbm, ones_v)
    plsc.subcore_barrier()

    def body(i, carry):
        off = w * epw + i * CH
        pltpu.sync_copy(src_hbm.at[pl.ds(off, CH)], idx_s)
        pltpu.sync_copy(dst_hbm.at[pl.ds(off, CH)], idx_d)
        pltpu.async_copy(x_hbm.at[idx_s], rows, sem).wait()
        pltpu.sync_copy(rows, acc.at[idx_d], add=True)
        pltpu.sync_copy(ones_v, dacc.at[idx_d], add=True)
        return carry

    lax.fori_loop(0, nchunks, body, 0)
    plsc.subcore_barrier()
    pltpu.sync_copy(acc.at[pl.ds(s * ZR, ZR)],
                    agg_hbm.at[c, pl.ds(s * ZR, ZR)])
    pltpu.sync_copy(dacc.at[pl.ds(s * ZR, ZR)],
                    deg_hbm.at[c, pl.ds(s * ZR, ZR)])


@functools.partial(
    pl.kernel,
    out_type=jax.ShapeDtypeStruct((NC, NP, D_IN), jnp.float32),
    mesh=_SC_MESH,
    scratch_types=[
        pltpu.VMEM((CH,), jnp.int32),
        pltpu.VMEM((CH,), jnp.int32),
        pltpu.VMEM((CH, D_IN), jnp.float32),
        pltpu.VMEM_SHARED((NP, D_IN), jnp.float32),
        pltpu.SemaphoreType.DMA,
    ],
)
def _sc_adj_s(lo_hbm, hi_hbm, src_hbm, dst_hbm, z_hbm, out_hbm,
              idx_s, idx_d, rows, acc, sem):
    c = lax.axis_index("c")
    s = lax.axis_index("s")
    eps = E // NS
    nchunks = eps // CH

    pltpu.sync_copy(z_hbm, acc.at[pl.ds(s * ZR, ZR)])
    plsc.subcore_barrier()

    def body(i, carry):
        off = s * eps + i * CH
        pltpu.sync_copy(src_hbm.at[pl.ds(off, CH)], idx_s)
        pltpu.sync_copy(dst_hbm.at[pl.ds(off, CH)], idx_d)

        @pl.when(c == 0)
        def _():
            pltpu.async_copy(lo_hbm.at[idx_s], rows, sem).wait()

        @pl.when(c == 1)
        def _():
            pltpu.async_copy(hi_hbm.at[idx_s], rows, sem).wait()

        pltpu.sync_copy(rows, acc.at[idx_d], add=True)
        return carry

    lax.fori_loop(0, nchunks, body, 0)
    plsc.subcore_barrier()
    pltpu.sync_copy(acc.at[pl.ds(s * ZR, ZR)],
                    out_hbm.at[c, pl.ds(s * ZR, ZR)])


_RB = 2000


def _tc_mlp_body(x_ref, a0_ref, a1_ref, d0_ref, d1_ref,
                 w1f_ref, b1f_ref, w2f_ref, b2f_ref,
                 w1p_ref, b1p_ref, w2p_ref, b2p_ref,
                 feat_ref, lo_ref, hi_ref):
    agg = a0_ref[0] + a1_ref[0]
    deg = d0_ref[...] + d1_ref[...]
    h = x_ref[...] + agg / jnp.maximum(deg, 1.0)

    hf = jnp.maximum(
        jax.lax.dot_general(h, w1f_ref[...], (((1,), (0,)), ((), ())),
                            preferred_element_type=jnp.float32) + b1f_ref[...],
        0.0)
    feat_ref[...] = jax.lax.dot_general(
        hf, w2f_ref[...], (((1,), (0,)), ((), ())),
        preferred_element_type=jnp.float32) + b2f_ref[...]

    hp = jnp.maximum(
        jax.lax.dot_general(h, w1p_ref[...], (((1,), (0,)), ((), ())),
                            preferred_element_type=jnp.float32) + b1p_ref[...],
        0.0)
    logits = jax.lax.dot_general(
        hp, w2p_ref[...], (((1,), (0,)), ((), ())),
        preferred_element_type=jnp.float32) + b2p_ref[...]

    m = jnp.max(logits, axis=-1, keepdims=True)
    ex = jnp.exp(logits - m)
    sm = ex / jnp.sum(ex, axis=-1, keepdims=True)
    lo_ref[...] = sm[:, :D_IN]
    hi_ref[...] = sm[:, D_IN:]


def _tc_mlp(x, agg_part, d0, d1, W1f, b1f, W2f, b2f, W1p, b1p, W2p, b2p):
    nb = N // _RB
    full = lambda r, cdim: pl.BlockSpec((r, cdim), lambda i: (0, 0))
    return pl.pallas_call(
        _tc_mlp_body,
        grid=(nb,),
        in_specs=[
            pl.BlockSpec((_RB, D_IN), lambda i: (i, 0)),
            pl.BlockSpec((1, _RB, D_IN), lambda i: (0, i, 0)),
            pl.BlockSpec((1, _RB, D_IN), lambda i: (1, i, 0)),
            pl.BlockSpec((_RB, 1), lambda i: (i, 0)),
            pl.BlockSpec((_RB, 1), lambda i: (i, 0)),
            full(D_IN, D_OUT), full(1, D_OUT), full(D_OUT, D_OUT), full(1, D_OUT),
            full(D_IN, A_DIM), full(1, A_DIM), full(A_DIM, A_DIM), full(1, A_DIM),
        ],
        out_specs=[
            pl.BlockSpec((_RB, D_OUT), lambda i: (i, 0)),
            pl.BlockSpec((_RB, D_IN), lambda i: (i, 0)),
            pl.BlockSpec((_RB, D_IN), lambda i: (i, 0)),
        ],
        out_shape=[
            jax.ShapeDtypeStruct((N, D_OUT), jnp.float32),
            jax.ShapeDtypeStruct((N, D_IN), jnp.float32),
            jax.ShapeDtypeStruct((N, D_IN), jnp.float32),
        ],
    )(x, agg_part, agg_part, d0, d1,
      W1f, b1f, W2f, b2f, W1p, b1p, W2p, b2p)


def _tc_pool_body(lo_ref, hi_ref, feat_ref, alo_ref, ahi_ref,
                  h_ref, adj_ref):
    i = pl.program_id(0)

    @pl.when(i == 0)
    def _():
        h_ref[...] = jnp.zeros_like(h_ref)
        adj_ref[...] = jnp.zeros_like(adj_ref)

    s_cat = jnp.concatenate([lo_ref[...], hi_ref[...]], axis=1)
    ct = (((0,), (0,)), ((), ()))
    h_ref[...] += jax.lax.dot_general(
        s_cat, feat_ref[...], ct, preferred_element_type=jnp.float32)
    adj_ref[:, :D_IN] += jax.lax.dot_general(
        s_cat, alo_ref[0], ct, preferred_element_type=jnp.float32)
    adj_ref[:, D_IN:] += jax.lax.dot_general(
        s_cat, ahi_ref[0], ct, preferred_element_type=jnp.float32)


def _tc_pool(s_lo, s_hi, feat, adj_part):
    nb = N // _RB
    return pl.pallas_call(
        _tc_pool_body,
        grid=(nb,),
        in_specs=[
            pl.BlockSpec((_RB, D_IN), lambda i: (i, 0)),
            pl.BlockSpec((_RB, D_IN), lambda i: (i, 0)),
            pl.BlockSpec((_RB, D_OUT), lambda i: (i, 0)),
            pl.BlockSpec((1, _RB, D_IN), lambda i: (0, i, 0)),
            pl.BlockSpec((1, _RB, D_IN), lambda i: (1, i, 0)),
        ],
        out_specs=[
            pl.BlockSpec((A_DIM, D_OUT), lambda i: (0, 0)),
            pl.BlockSpec((A_DIM, A_DIM), lambda i: (0, 0)),
        ],
        out_shape=[
            jax.ShapeDtypeStruct((A_DIM, D_OUT), jnp.float32),
            jax.ShapeDtypeStruct((A_DIM, A_DIM), jnp.float32),
        ],
    )(s_lo, s_hi, feat, adj_part, adj_part)


def kernel(x, edge_index, W1f, b1f, W2f, b2f, W1p, b1p, W2p, b2p):
    src = edge_index[0]
    dst = edge_index[1]
    z = jnp.zeros((ZR, D_IN), jnp.float32)
    z1 = jnp.zeros((ZR,), jnp.float32)
    ones = jnp.ones((CH,), jnp.float32)

    agg_part, deg_part = _sc_gin_agg(x, src, dst, z, z1, ones)
    d0 = deg_part[0].reshape(NP, 1)
    d1 = deg_part[1].reshape(NP, 1)

    feat, s_lo, s_hi = _tc_mlp(
        x, agg_part, d0, d1,
        W1f, b1f.reshape(1, D_OUT), W2f, b2f.reshape(1, D_OUT),
        W1p, b1p.reshape(1, A_DIM), W2p, b2p.reshape(1, A_DIM))

    adj_part = _sc_adj_s(s_lo, s_hi, src, dst, z)

    h_out, adj_new = _tc_pool(s_lo, s_hi, feat, adj_part)
    return (adj_new, h_out)


# speedup vs baseline: 9.4309x; 2.3128x over previous
import functools

import jax
import jax.numpy as jnp
from jax import lax
from jax.experimental import pallas as pl
from jax.experimental.pallas import tpu as pltpu
from jax.experimental.pallas import tpu_sc as plsc

N = 10000
E = 320000
D_IN = 128
D_OUT = 128
A_DIM = 256

NC = 2
NS = 16
NW = NC * NS
NP = 10240
ZR = NP // NS
CH = 128
EPW = 10240
EPAD = NW * EPW
NCHA = EPW // CH
NCHC = (EPAD // NS) // CH

_SC_MESH = plsc.VectorSubcoreMesh(core_axis_name="c", subcore_axis_name="s")


@functools.partial(
    pl.kernel,
    out_type=[
        jax.ShapeDtypeStruct((NC, NP, D_IN), jnp.float32),
        jax.ShapeDtypeStruct((NC, NP), jnp.float32),
    ],
    mesh=_SC_MESH,
    scratch_types=[
        (pltpu.VMEM((CH,), jnp.int32),) * 2,
        (pltpu.VMEM((CH,), jnp.int32),) * 2,
        (pltpu.VMEM((CH, D_IN), jnp.float32),) * 2,
        pltpu.VMEM((CH,), jnp.float32),
        pltpu.VMEM_SHARED((NP, D_IN), jnp.float32),
        pltpu.VMEM_SHARED((NP,), jnp.float32),
        (pltpu.SemaphoreType.DMA,) * 2,
        (pltpu.SemaphoreType.DMA,) * 2,
    ],
)
def _sc_gin_agg(x_hbm, src_hbm, dst_hbm, z_hbm, z1_hbm, ones_hbm,
                agg_hbm, deg_hbm,
                idx_s, idx_d, rows, ones_v, acc, dacc, isem, gsem):
    c = lax.axis_index("c")
    s = lax.axis_index("s")
    w = c * NS + s
    base = w * EPW

    pltpu.sync_copy(z_hbm, acc.at[pl.ds(s * ZR, ZR)])
    pltpu.sync_copy(z1_hbm, dacc.at[pl.ds(s * ZR, ZR)])
    pltpu.sync_copy(ones_hbm, ones_v)
    plsc.subcore_barrier()

    def load_idx(i, b):
        off = base + i * CH
        pltpu.async_copy(src_hbm.at[pl.ds(off, CH)], idx_s[b], isem[b])
        pltpu.async_copy(dst_hbm.at[pl.ds(off, CH)], idx_d[b], isem[b])

    def wait_idx(i, b):
        off = base + i * CH
        pltpu.make_async_copy(src_hbm.at[pl.ds(off, CH)], idx_s[b],
                              isem[b]).wait()
        pltpu.make_async_copy(dst_hbm.at[pl.ds(off, CH)], idx_d[b],
                              isem[b]).wait()

    load_idx(0, 0)
    load_idx(1, 1)
    wait_idx(0, 0)
    pltpu.async_copy(x_hbm.at[idx_s[0]], rows[0], gsem[0])

    def step(i, b):
        nxt = 1 - b

        @pl.when(i + 1 < NCHA)
        def _():
            wait_idx(i + 1, nxt)
            pltpu.async_copy(x_hbm.at[idx_s[nxt]], rows[nxt], gsem[nxt])

        pltpu.make_async_copy(x_hbm.at[idx_s[b]], rows[b], gsem[b]).wait()
        pltpu.sync_copy(rows[b], acc.at[idx_d[b]], add=True)
        pltpu.sync_copy(ones_v, dacc.at[idx_d[b]], add=True)

        @pl.when(i + 2 < NCHA)
        def _():
            load_idx(i + 2, b)

    def outer(g, carry):
        step(2 * g, 0)
        step(2 * g + 1, 1)
        return carry

    lax.fori_loop(0, NCHA // 2, outer, 0)

    plsc.subcore_barrier()
    pltpu.sync_copy(acc.at[pl.ds(s * ZR, ZR)],
                    agg_hbm.at[c, pl.ds(s * ZR, ZR)])
    pltpu.sync_copy(dacc.at[pl.ds(s * ZR, ZR)],
                    deg_hbm.at[c, pl.ds(s * ZR, ZR)])


@functools.partial(
    pl.kernel,
    out_type=jax.ShapeDtypeStruct((NC, NP, D_IN), jnp.float32),
    mesh=_SC_MESH,
    scratch_types=[
        (pltpu.VMEM((CH,), jnp.int32),) * 2,
        (pltpu.VMEM((CH,), jnp.int32),) * 2,
        (pltpu.VMEM((CH, D_IN), jnp.float32),) * 2,
        pltpu.VMEM_SHARED((NP, D_IN), jnp.float32),
        (pltpu.SemaphoreType.DMA,) * 2,
        (pltpu.SemaphoreType.DMA,) * 2,
    ],
)
def _sc_adj_s(lo_hbm, hi_hbm, src_hbm, dst_hbm, z_hbm, out_hbm,
              idx_s, idx_d, rows, acc, isem, gsem):
    c = lax.axis_index("c")
    s = lax.axis_index("s")
    eps = EPAD // NS
    base = s * eps

    pltpu.sync_copy(z_hbm, acc.at[pl.ds(s * ZR, ZR)])
    plsc.subcore_barrier()

    def load_idx(i, b):
        off = base + i * CH
        pltpu.async_copy(src_hbm.at[pl.ds(off, CH)], idx_s[b], isem[b])
        pltpu.async_copy(dst_hbm.at[pl.ds(off, CH)], idx_d[b], isem[b])

    def wait_idx(i, b):
        off = base + i * CH
        pltpu.make_async_copy(src_hbm.at[pl.ds(off, CH)], idx_s[b],
                              isem[b]).wait()
        pltpu.make_async_copy(dst_hbm.at[pl.ds(off, CH)], idx_d[b],
                              isem[b]).wait()

    def start_gather(b):
        @pl.when(c == 0)
        def _():
            pltpu.async_copy(lo_hbm.at[idx_s[b]], rows[b], gsem[b])

        @pl.when(c == 1)
        def _():
            pltpu.async_copy(hi_hbm.at[idx_s[b]], rows[b], gsem[b])

    load_idx(0, 0)
    load_idx(1, 1)
    wait_idx(0, 0)
    start_gather(0)

    def step(i, b):
        nxt = 1 - b

        @pl.when(i + 1 < NCHC)
        def _():
            wait_idx(i + 1, nxt)
            start_gather(nxt)

        pltpu.make_async_copy(lo_hbm.at[idx_s[b]], rows[b], gsem[b]).wait()
        pltpu.sync_copy(rows[b], acc.at[idx_d[b]], add=True)

        @pl.when(i + 2 < NCHC)
        def _():
            load_idx(i + 2, b)

    def outer(g, carry):
        step(2 * g, 0)
        step(2 * g + 1, 1)
        return carry

    lax.fori_loop(0, NCHC // 2, outer, 0)

    plsc.subcore_barrier()
    pltpu.sync_copy(acc.at[pl.ds(s * ZR, ZR)],
                    out_hbm.at[c, pl.ds(s * ZR, ZR)])


_RB = 2000


def _tc_mlp_body(x_ref, a0_ref, a1_ref, d0_ref, d1_ref,
                 w1f_ref, b1f_ref, w2f_ref, b2f_ref,
                 w1p_ref, b1p_ref, w2p_ref, b2p_ref,
                 feat_ref, lo_ref, hi_ref):
    agg = a0_ref[0] + a1_ref[0]
    deg = d0_ref[...] + d1_ref[...]
    h = x_ref[...] + agg / jnp.maximum(deg, 1.0)

    hf = jnp.maximum(
        jax.lax.dot_general(h, w1f_ref[...], (((1,), (0,)), ((), ())),
                            preferred_element_type=jnp.float32) + b1f_ref[...],
        0.0)
    feat_ref[...] = jax.lax.dot_general(
        hf, w2f_ref[...], (((1,), (0,)), ((), ())),
        preferred_element_type=jnp.float32) + b2f_ref[...]

    hp = jnp.maximum(
        jax.lax.dot_general(h, w1p_ref[...], (((1,), (0,)), ((), ())),
                            preferred_element_type=jnp.float32) + b1p_ref[...],
        0.0)
    logits = jax.lax.dot_general(
        hp, w2p_ref[...], (((1,), (0,)), ((), ())),
        preferred_element_type=jnp.float32) + b2p_ref[...]

    m = jnp.max(logits, axis=-1, keepdims=True)
    ex = jnp.exp(logits - m)
    sm = ex / jnp.sum(ex, axis=-1, keepdims=True)
    lo_ref[...] = sm[:, :D_IN]
    hi_ref[...] = sm[:, D_IN:]


def _tc_mlp(x, agg_part, d0, d1, W1f, b1f, W2f, b2f, W1p, b1p, W2p, b2p):
    nb = N // _RB
    full = lambda r, cdim: pl.BlockSpec((r, cdim), lambda i: (0, 0))
    return pl.pallas_call(
        _tc_mlp_body,
        grid=(nb,),
        in_specs=[
            pl.BlockSpec((_RB, D_IN), lambda i: (i, 0)),
            pl.BlockSpec((1, _RB, D_IN), lambda i: (0, i, 0)),
            pl.BlockSpec((1, _RB, D_IN), lambda i: (1, i, 0)),
            pl.BlockSpec((_RB, 1), lambda i: (i, 0)),
            pl.BlockSpec((_RB, 1), lambda i: (i, 0)),
            full(D_IN, D_OUT), full(1, D_OUT), full(D_OUT, D_OUT), full(1, D_OUT),
            full(D_IN, A_DIM), full(1, A_DIM), full(A_DIM, A_DIM), full(1, A_DIM),
        ],
        out_specs=[
            pl.BlockSpec((_RB, D_OUT), lambda i: (i, 0)),
            pl.BlockSpec((_RB, D_IN), lambda i: (i, 0)),
            pl.BlockSpec((_RB, D_IN), lambda i: (i, 0)),
        ],
        out_shape=[
            jax.ShapeDtypeStruct((N, D_OUT), jnp.float32),
            jax.ShapeDtypeStruct((N, D_IN), jnp.float32),
            jax.ShapeDtypeStruct((N, D_IN), jnp.float32),
        ],
    )(x, agg_part, agg_part, d0, d1,
      W1f, b1f, W2f, b2f, W1p, b1p, W2p, b2p)


def _tc_pool_body(lo_ref, hi_ref, feat_ref, alo_ref, ahi_ref,
                  h_ref, adj_ref):
    i = pl.program_id(0)

    @pl.when(i == 0)
    def _():
        h_ref[...] = jnp.zeros_like(h_ref)
        adj_ref[...] = jnp.zeros_like(adj_ref)

    s_cat = jnp.concatenate([lo_ref[...], hi_ref[...]], axis=1)
    ct = (((0,), (0,)), ((), ()))
    h_ref[...] += jax.lax.dot_general(
        s_cat, feat_ref[...], ct, preferred_element_type=jnp.float32)
    adj_ref[:, :D_IN] += jax.lax.dot_general(
        s_cat, alo_ref[0], ct, preferred_element_type=jnp.float32)
    adj_ref[:, D_IN:] += jax.lax.dot_general(
        s_cat, ahi_ref[0], ct, preferred_element_type=jnp.float32)


def _tc_pool(s_lo, s_hi, feat, adj_part):
    nb = N // _RB
    return pl.pallas_call(
        _tc_pool_body,
        grid=(nb,),
        in_specs=[
            pl.BlockSpec((_RB, D_IN), lambda i: (i, 0)),
            pl.BlockSpec((_RB, D_IN), lambda i: (i, 0)),
            pl.BlockSpec((_RB, D_OUT), lambda i: (i, 0)),
            pl.BlockSpec((1, _RB, D_IN), lambda i: (0, i, 0)),
            pl.BlockSpec((1, _RB, D_IN), lambda i: (1, i, 0)),
        ],
        out_specs=[
            pl.BlockSpec((A_DIM, D_OUT), lambda i: (0, 0)),
            pl.BlockSpec((A_DIM, A_DIM), lambda i: (0, 0)),
        ],
        out_shape=[
            jax.ShapeDtypeStruct((A_DIM, D_OUT), jnp.float32),
            jax.ShapeDtypeStruct((A_DIM, A_DIM), jnp.float32),
        ],
    )(s_lo, s_hi, feat, adj_part, adj_part)


def kernel(x, edge_index, W1f, b1f, W2f, b2f, W1p, b1p, W2p, b2p):
    pad_iota = jnp.arange(EPAD - E, dtype=jnp.int32)
    src = jnp.concatenate([edge_index[0], pad_iota % N])
    dst = jnp.concatenate([edge_index[1], N + (pad_iota % (NP - N))])
    z = jnp.zeros((ZR, D_IN), jnp.float32)
    z1 = jnp.zeros((ZR,), jnp.float32)
    ones = jnp.ones((CH,), jnp.float32)

    agg_part, deg_part = _sc_gin_agg(x, src, dst, z, z1, ones)
    d0 = deg_part[0].reshape(NP, 1)
    d1 = deg_part[1].reshape(NP, 1)

    feat, s_lo, s_hi = _tc_mlp(
        x, agg_part, d0, d1,
        W1f, b1f.reshape(1, D_OUT), W2f, b2f.reshape(1, D_OUT),
        W1p, b1p.reshape(1, A_DIM), W2p, b2p.reshape(1, A_DIM))

    adj_part = _sc_adj_s(s_lo, s_hi, src, dst, z)

    h_out, adj_new = _tc_pool(s_lo, s_hi, feat, adj_part)
    return (adj_new, h_out)
